# Initial kernel scaffold; baseline (speedup 1.0000x reference)
import jax
import jax.numpy as jnp
from jax.experimental import pallas as pl


def kernel(x, edge_index, W1, a_src1, a_dst1, b1, W2, a_src2, a_dst2, b2):
    raise NotImplementedError("write your pallas kernel here")


# speedup vs baseline: 1.1596x; 1.1596x over previous
import functools

import jax
import jax.numpy as jnp
from jax.experimental import pallas as pl

N = 10000
E = 320000
D = 128
H1 = 8
C1 = 8
HID = H1 * C1
C2 = 64


def _log_softmax_body(x_ref, o_ref):
    v = x_ref[...]
    m = jnp.max(v, axis=1, keepdims=True)
    z = v - m
    o_ref[...] = z - jnp.log(jnp.sum(jnp.exp(z), axis=1, keepdims=True))


def _gat_conv(h_in, edge_index, W, a_src, a_dst, heads, out_ch):
    n = h_in.shape[0]
    h = (h_in @ W).reshape(n, heads, out_ch)
    src = edge_index[0]
    dst = edge_index[1]
    alpha_src = (h * a_src[None, :, :]).sum(-1)
    alpha_dst = (h * a_dst[None, :, :]).sum(-1)
    e = alpha_src[src] + alpha_dst[dst]
    e = jax.nn.leaky_relu(e, negative_slope=0.2)
    ex = jnp.exp(e)
    denom = jax.ops.segment_sum(ex, dst, num_segments=n)
    msg = h[src] * ex[:, :, None]
    out = jax.ops.segment_sum(msg, dst, num_segments=n)
    return out / (denom[:, :, None] + 1e-16)


def kernel(x, edge_index, W1, a_src1, a_dst1, b1, W2, a_src2, a_dst2, b2):
    h1 = _gat_conv(x, edge_index, W1, a_src1, a_dst1, H1, C1)
    h1 = h1.reshape(N, HID) + b1
    h1 = jax.nn.elu(h1)
    h2 = _gat_conv(h1, edge_index, W2, a_src2, a_dst2, 1, C2)
    h2 = h2.mean(axis=1) + b2
    blk = 1000
    return pl.pallas_call(
        _log_softmax_body,
        grid=(N // blk,),
        in_specs=[pl.BlockSpec((blk, C2), lambda i: (i, 0))],
        out_specs=pl.BlockSpec((blk, C2), lambda i: (i, 0)),
        out_shape=jax.ShapeDtypeStruct((N, C2), jnp.float32),
    )(h2)


# speedup vs baseline: 41.0922x; 35.4368x over previous
import functools

import numpy as np

import jax
import jax.numpy as jnp
from jax import lax
from jax.experimental import pallas as pl
from jax.experimental.pallas import tpu as pltpu
from jax.experimental.pallas import tpu_sc as plsc

N = 10000
E = 320000
D = 128
HID = 64
C2 = 64

NC = 2
NS = 16
NW = NC * NS
EPW = E // NW
SUB = 125
NCHUNK = EPW // SUB
ROWW = 80
MC = 64
ACCW = 72
NPAD = 10112
NPT = NPAD // NS
NROWZ = 79

_ROWBLK = 1000
_GRID = N // _ROWBLK


def _make_edge_kernel(nheads):
    mesh = plsc.VectorSubcoreMesh(core_axis_name="c", subcore_axis_name="s")

    @functools.partial(
        pl.kernel,
        out_type=jax.ShapeDtypeStruct((NC, NPAD, ACCW), jnp.float32),
        mesh=mesh,
        scratch_types=(
            pltpu.VMEM((1, SUB), jnp.int32),
            pltpu.VMEM((1, SUB), jnp.int32),
            pltpu.VMEM((SUB, ROWW), jnp.float32),
            pltpu.VMEM((SUB, 16), jnp.float32),
            pltpu.VMEM((SUB, ACCW), jnp.float32),
            pltpu.VMEM_SHARED((NPAD, ACCW), jnp.float32),
            pltpu.SemaphoreType.DMA,
            pltpu.SemaphoreType.DMA,
        ),
        compiler_params=pltpu.CompilerParams(use_tc_tiling_on_sc=False,
                                             needs_layout_passes=False),
    )
    def edge_kernel(htab, adtab, src_i, dst_i, acc_o,
                    srcv, dstv, g1, g2, msgex, acc_s, sem1, sem2):
        c = lax.axis_index("c")
        s = lax.axis_index("s")
        wid = s * NC + c
        z16 = jnp.zeros((16,), jnp.float32)
        iot = lax.iota(jnp.int32, 16)

        def zrow(r, carry):
            for k in (0, 16, 32, 48, 56):
                msgex[r, pl.ds(k, 16)] = z16
            return carry

        lax.fori_loop(0, NROWZ, zrow, 0)
        for t in range(NPT // NROWZ):
            pltpu.sync_copy(msgex.at[pl.ds(0, NROWZ)],
                            acc_s.at[pl.ds(s * NPT + t * NROWZ, NROWZ)])
        plsc.subcore_barrier()

        def chunk_body(ch, carry):
            rowbase = wid * NCHUNK + ch
            pltpu.sync_copy(src_i.at[pl.ds(rowbase, 1)], srcv)
            pltpu.sync_copy(dst_i.at[pl.ds(rowbase, 1)], dstv)
            d1 = pltpu.async_copy(htab.at[srcv.at[0]], g1, sem1)
            d2 = pltpu.async_copy(adtab.at[dstv.at[0]], g2, sem2)
            d1.wait()
            d2.wait()

            def edge_body(e, ecarry):
                a = g1[e, pl.ds(MC, 16)]
                b = g2[e, pl.ds(0, 16)]
                es = a + b
                es = jnp.maximum(es, es * jnp.float32(0.2))
                ex = jnp.exp(es)
                erow = jnp.full((16,), e, jnp.int32)
                iotl = lax.iota(jnp.int32, 16)
                plsc.store_scatter(msgex, [erow, MC + (iotl % 8)], ex,
                                   mask=iotl < 8)
                for v in range(MC // 16):
                    if nheads == 8:
                        pv = MC + 2 * v + (iotl // 8)
                    else:
                        pv = MC + 0 * (iotl // 8)
                    hx = g1[e, pl.ds(16 * v, 16)]
                    exv = plsc.load_gather(msgex, [erow, pv])
                    msgex[e, pl.ds(16 * v, 16)] = hx * exv
                return ecarry

            lax.fori_loop(0, SUB, edge_body, 0)

            pltpu.sync_copy(msgex.at[pl.ds(0, SUB)],
                            acc_s.at[dstv.at[0]], add=True)
            return carry

        lax.fori_loop(0, NCHUNK, chunk_body, 0)
        plsc.subcore_barrier()

        pltpu.sync_copy(acc_s.at[pl.ds(s * NPT, NPT)],
                        acc_o.at[c, pl.ds(s * NPT, NPT)])

    return edge_kernel


_edge_kernel_l1 = _make_edge_kernel(8)
_edge_kernel_l2 = _make_edge_kernel(1)


def _tc1_body(x_ref, w1_ref, asd_ref, ad_ref, htab_ref, adtab_ref):
    h = jnp.dot(x_ref[...], w1_ref[...], preferred_element_type=jnp.float32)
    sa = jnp.dot(h, asd_ref[...], preferred_element_type=jnp.float32)
    htab_ref[...] = jnp.concatenate([h, sa], axis=1)
    adtab_ref[...] = jnp.dot(h, ad_ref[...], preferred_element_type=jnp.float32)


def _tc2_body(acc_ref, erep_ref, b1_ref, w2_ref, a2sd_ref, a2d_ref,
              htab_ref, adtab_ref):
    both = acc_ref[0] + acc_ref[1]
    acc = both[:, :MC]
    den = both[:, MC:ACCW]
    dex = jnp.dot(den, erep_ref[...], preferred_element_type=jnp.float32)
    h1 = acc / (dex + 1e-16) + b1_ref[...]
    h1 = jnp.where(h1 > 0, h1, jnp.exp(jnp.minimum(h1, 0.0)) - 1.0)
    h2 = jnp.dot(h1, w2_ref[...], preferred_element_type=jnp.float32)
    sa = jnp.dot(h2, a2sd_ref[...], preferred_element_type=jnp.float32)
    htab_ref[...] = jnp.concatenate([h2, sa], axis=1)
    adtab_ref[...] = jnp.dot(h2, a2d_ref[...], preferred_element_type=jnp.float32)


def _tc3_body(acc_ref, bmat_ref, b2_ref, out_ref):
    both = acc_ref[0] + acc_ref[1]
    acc = both[:, :MC]
    den = both[:, MC:ACCW]
    dex = jnp.dot(den, bmat_ref[...], preferred_element_type=jnp.float32)
    v = acc / (dex + 1e-16) + b2_ref[...]
    m = jnp.max(v, axis=1, keepdims=True)
    z = v - m
    out_ref[...] = z - jnp.log(jnp.sum(jnp.exp(z), axis=1, keepdims=True))


def _full(shape):
    return pl.BlockSpec(shape, lambda i: tuple(0 for _ in shape))


def kernel(x, edge_index, W1, a_src1, a_dst1, b1, W2, a_src2, a_dst2, b2):
    f32 = jnp.float32
    src2d = edge_index[0].reshape(E // SUB, SUB)
    dst2d = edge_index[1].reshape(E // SUB, SUB)

    eye8 = jnp.eye(8, dtype=f32)
    as64 = (a_src1[:, :, None] * eye8[:, None, :]).reshape(64, 8)
    ad64 = (a_dst1[:, :, None] * eye8[:, None, :]).reshape(64, 8)
    asd = jnp.concatenate([as64, ad64], axis=1)
    ad16 = jnp.concatenate([ad64, jnp.zeros((64, 8), f32)], axis=1)
    erep = jnp.repeat(jnp.eye(8, dtype=f32), 8, axis=1)
    a2sd = jnp.concatenate([a_src2.T, jnp.zeros((64, 15), f32)], axis=1)
    a2d = jnp.concatenate([a_dst2.T, jnp.zeros((64, 15), f32)], axis=1)
    bmat = jnp.concatenate([jnp.ones((1, 64), f32),
                            jnp.zeros((7, 64), f32)], axis=0)
    b1r = b1.reshape(1, HID)
    b2r = b2.reshape(1, C2)

    htab1, adtab1 = pl.pallas_call(
        _tc1_body,
        grid=(_GRID,),
        in_specs=[
            pl.BlockSpec((_ROWBLK, D), lambda i: (i, 0)),
            _full((D, HID)),
            _full((64, 16)),
            _full((64, 16)),
        ],
        out_specs=[
            pl.BlockSpec((_ROWBLK, ROWW), lambda i: (i, 0)),
            pl.BlockSpec((_ROWBLK, 16), lambda i: (i, 0)),
        ],
        out_shape=[
            jax.ShapeDtypeStruct((N, ROWW), f32),
            jax.ShapeDtypeStruct((N, 16), f32),
        ],
    )(x, W1, asd, ad16)

    acc1 = _edge_kernel_l1(htab1, adtab1, src2d, dst2d)[:, :N]

    htab2, adtab2 = pl.pallas_call(
        _tc2_body,
        grid=(_GRID,),
        in_specs=[
            pl.BlockSpec((NC, _ROWBLK, ACCW), lambda i: (0, i, 0)),
            _full((8, 64)),
            _full((1, HID)),
            _full((HID, C2)),
            _full((64, 16)),
            _full((64, 16)),
        ],
        out_specs=[
            pl.BlockSpec((_ROWBLK, ROWW), lambda i: (i, 0)),
            pl.BlockSpec((_ROWBLK, 16), lambda i: (i, 0)),
        ],
        out_shape=[
            jax.ShapeDtypeStruct((N, ROWW), f32),
            jax.ShapeDtypeStruct((N, 16), f32),
        ],
    )(acc1, erep, b1r, W2, a2sd, a2d)

    acc2 = _edge_kernel_l2(htab2, adtab2, src2d, dst2d)[:, :N]

    out = pl.pallas_call(
        _tc3_body,
        grid=(_GRID,),
        in_specs=[
            pl.BlockSpec((NC, _ROWBLK, ACCW), lambda i: (0, i, 0)),
            _full((8, 64)),
            _full((1, C2)),
        ],
        out_specs=pl.BlockSpec((_ROWBLK, C2), lambda i: (i, 0)),
        out_shape=jax.ShapeDtypeStruct((N, C2), f32),
    )(acc2, bmat, b2r)

    return out


# speedup vs baseline: 54.3565x; 1.3228x over previous
import functools

import numpy as np

import jax
import jax.numpy as jnp
from jax import lax
from jax.experimental import pallas as pl
from jax.experimental.pallas import tpu as pltpu
from jax.experimental.pallas import tpu_sc as plsc

N = 10000
E = 320000
D = 128
HID = 64
C2 = 64

NC = 2
NS = 16
NW = NC * NS
EPW = E // NW
SUB = 125
NSUB = 2
CHUNK = SUB * NSUB
NCHUNK = EPW // CHUNK
ROWW = 80
MC = 64
ACCW = 72
NPAD = 10112
NPT = NPAD // NS
NROWZ = 79

_ROWBLK = 1000
_GRID = N // _ROWBLK


def _make_edge_kernel(nheads):
    mesh = plsc.VectorSubcoreMesh(core_axis_name="c", subcore_axis_name="s")

    @functools.partial(
        pl.kernel,
        out_type=jax.ShapeDtypeStruct((NC, NPAD, ACCW), jnp.float32),
        mesh=mesh,
        scratch_types=(
            pltpu.VMEM((NSUB, SUB), jnp.int32),
            pltpu.VMEM((NSUB, SUB), jnp.int32),
            pltpu.VMEM((NSUB, SUB), jnp.int32),
            pltpu.VMEM((NSUB, SUB), jnp.int32),
            pltpu.VMEM((CHUNK, ROWW), jnp.float32),
            pltpu.VMEM((CHUNK, ROWW), jnp.float32),
            pltpu.VMEM((CHUNK, 16), jnp.float32),
            pltpu.VMEM((CHUNK, 16), jnp.float32),
            pltpu.VMEM((CHUNK, ACCW), jnp.float32),
            pltpu.VMEM_SHARED((NPAD, ACCW), jnp.float32),
            pltpu.SemaphoreType.DMA,
            pltpu.SemaphoreType.DMA,
            pltpu.SemaphoreType.DMA,
            pltpu.SemaphoreType.DMA,
        ),
        compiler_params=pltpu.CompilerParams(use_tc_tiling_on_sc=False,
                                             needs_layout_passes=False),
    )
    def edge_kernel(htab, adtab, src_i, dst_i, acc_o,
                    srcv0, srcv1, dstv0, dstv1, g1a, g1b, g2a, g2b,
                    msgex, acc_s, s1a, s1b, s2a, s2b):
        c = lax.axis_index("c")
        s = lax.axis_index("s")
        wid = s * NC + c
        z16 = jnp.zeros((16,), jnp.float32)
        bufs = ((srcv0, dstv0, g1a, g2a, s1a, s2a),
                (srcv1, dstv1, g1b, g2b, s1b, s2b))

        def fire(ch, b):
            si, di, g1, g2, sh, sa = bufs[b]
            rowbase = wid * (EPW // SUB) + ch * NSUB
            pltpu.sync_copy(src_i.at[pl.ds(rowbase, NSUB)], si)
            pltpu.sync_copy(dst_i.at[pl.ds(rowbase, NSUB)], di)
            for j in range(NSUB):
                pltpu.async_copy(htab.at[si.at[j]],
                                 g1.at[pl.ds(j * SUB, SUB)], sh)
                pltpu.async_copy(adtab.at[di.at[j]],
                                 g2.at[pl.ds(j * SUB, SUB)], sa)

        def drain(b):
            si, di, g1, g2, sh, sa = bufs[b]
            for j in range(NSUB):
                pltpu.make_async_copy(htab.at[si.at[j]],
                                      g1.at[pl.ds(j * SUB, SUB)], sh).wait()
                pltpu.make_async_copy(adtab.at[di.at[j]],
                                      g2.at[pl.ds(j * SUB, SUB)], sa).wait()

        def zrow(r, carry):
            for k in (0, 16, 32, 48, 56):
                msgex[r, pl.ds(k, 16)] = z16
            return carry

        lax.fori_loop(0, NROWZ, zrow, 0)
        for t in range(NPT // NROWZ):
            pltpu.sync_copy(msgex.at[pl.ds(0, NROWZ)],
                            acc_s.at[pl.ds(s * NPT + t * NROWZ, NROWZ)])
        plsc.subcore_barrier()

        gdn = lax.GatherDimensionNumbers(
            offset_dims=(), collapsed_slice_dims=(0,), start_index_map=(0,))

        def compute_scatter(b):
            si, di, g1, g2, sh, sa = bufs[b]

            def edge_body(e, ecarry):
                av = g1[e, pl.ds(MC, 16)]
                bv = g2[e, pl.ds(0, 16)]
                es = av + bv
                es = jnp.maximum(es, es * jnp.float32(0.2))
                ex = jnp.exp(es)
                erow = jnp.full((16,), e, jnp.int32)
                iotl = lax.iota(jnp.int32, 16)
                plsc.store_scatter(msgex, [erow, MC + (iotl % 8)], ex,
                                   mask=iotl < 8)
                for v in range(MC // 16):
                    if nheads == 8:
                        pv = 2 * v + (iotl // 8)
                    else:
                        pv = 0 * (iotl // 8)
                    hx = g1[e, pl.ds(16 * v, 16)]
                    exv = lax.gather(
                        ex, pv[:, None], gdn, (1,),
                        mode=lax.GatherScatterMode.PROMISE_IN_BOUNDS)
                    msgex[e, pl.ds(16 * v, 16)] = hx * exv
                return ecarry

            lax.fori_loop(0, CHUNK, edge_body, 0, unroll=4)
            for j in range(NSUB):
                pltpu.sync_copy(msgex.at[pl.ds(j * SUB, SUB)],
                                acc_s.at[di.at[j]], add=True)

        fire(0, 0)

        def pair_body(i, carry):
            fire(2 * i + 1, 1)
            drain(0)
            compute_scatter(0)

            @pl.when(i < NCHUNK // 2 - 1)
            def _():
                fire(2 * i + 2, 0)

            drain(1)
            compute_scatter(1)
            return carry

        lax.fori_loop(0, NCHUNK // 2, pair_body, 0)
        plsc.subcore_barrier()

        pltpu.sync_copy(acc_s.at[pl.ds(s * NPT, NPT)],
                        acc_o.at[c, pl.ds(s * NPT, NPT)])

    return edge_kernel


_edge_kernel_l1 = _make_edge_kernel(8)
_edge_kernel_l2 = _make_edge_kernel(1)


def _tc1_body(x_ref, w1_ref, asd_ref, ad_ref, htab_ref, adtab_ref):
    h = jnp.dot(x_ref[...], w1_ref[...], preferred_element_type=jnp.float32)
    sa = jnp.dot(h, asd_ref[...], preferred_element_type=jnp.float32)
    htab_ref[...] = jnp.concatenate([h, sa], axis=1)
    adtab_ref[...] = jnp.dot(h, ad_ref[...], preferred_element_type=jnp.float32)


def _tc2_body(acc_ref, erep_ref, b1_ref, w2_ref, a2sd_ref, a2d_ref,
              htab_ref, adtab_ref):
    both = acc_ref[0] + acc_ref[1]
    acc = both[:, :MC]
    den = both[:, MC:ACCW]
    dex = jnp.dot(den, erep_ref[...], preferred_element_type=jnp.float32)
    h1 = acc / (dex + 1e-16) + b1_ref[...]
    h1 = jnp.where(h1 > 0, h1, jnp.exp(jnp.minimum(h1, 0.0)) - 1.0)
    h2 = jnp.dot(h1, w2_ref[...], preferred_element_type=jnp.float32)
    sa = jnp.dot(h2, a2sd_ref[...], preferred_element_type=jnp.float32)
    htab_ref[...] = jnp.concatenate([h2, sa], axis=1)
    adtab_ref[...] = jnp.dot(h2, a2d_ref[...], preferred_element_type=jnp.float32)


def _tc3_body(acc_ref, bmat_ref, b2_ref, out_ref):
    both = acc_ref[0] + acc_ref[1]
    acc = both[:, :MC]
    den = both[:, MC:ACCW]
    dex = jnp.dot(den, bmat_ref[...], preferred_element_type=jnp.float32)
    v = acc / (dex + 1e-16) + b2_ref[...]
    m = jnp.max(v, axis=1, keepdims=True)
    z = v - m
    out_ref[...] = z - jnp.log(jnp.sum(jnp.exp(z), axis=1, keepdims=True))


def _full(shape):
    return pl.BlockSpec(shape, lambda i: tuple(0 for _ in shape))


def kernel(x, edge_index, W1, a_src1, a_dst1, b1, W2, a_src2, a_dst2, b2):
    f32 = jnp.float32
    src2d = edge_index[0].reshape(E // SUB, SUB)
    dst2d = edge_index[1].reshape(E // SUB, SUB)

    eye8 = jnp.eye(8, dtype=f32)
    as64 = (a_src1[:, :, None] * eye8[:, None, :]).reshape(64, 8)
    ad64 = (a_dst1[:, :, None] * eye8[:, None, :]).reshape(64, 8)
    asd = jnp.concatenate([as64, ad64], axis=1)
    ad16 = jnp.concatenate([ad64, jnp.zeros((64, 8), f32)], axis=1)
    erep = jnp.repeat(jnp.eye(8, dtype=f32), 8, axis=1)
    a2sd = jnp.concatenate([a_src2.T, jnp.zeros((64, 15), f32)], axis=1)
    a2d = jnp.concatenate([a_dst2.T, jnp.zeros((64, 15), f32)], axis=1)
    bmat = jnp.concatenate([jnp.ones((1, 64), f32),
                            jnp.zeros((7, 64), f32)], axis=0)
    b1r = b1.reshape(1, HID)
    b2r = b2.reshape(1, C2)

    htab1, adtab1 = pl.pallas_call(
        _tc1_body,
        grid=(_GRID,),
        in_specs=[
            pl.BlockSpec((_ROWBLK, D), lambda i: (i, 0)),
            _full((D, HID)),
            _full((64, 16)),
            _full((64, 16)),
        ],
        out_specs=[
            pl.BlockSpec((_ROWBLK, ROWW), lambda i: (i, 0)),
            pl.BlockSpec((_ROWBLK, 16), lambda i: (i, 0)),
        ],
        out_shape=[
            jax.ShapeDtypeStruct((N, ROWW), f32),
            jax.ShapeDtypeStruct((N, 16), f32),
        ],
    )(x, W1, asd, ad16)

    acc1 = _edge_kernel_l1(htab1, adtab1, src2d, dst2d)[:, :N]

    htab2, adtab2 = pl.pallas_call(
        _tc2_body,
        grid=(_GRID,),
        in_specs=[
            pl.BlockSpec((NC, _ROWBLK, ACCW), lambda i: (0, i, 0)),
            _full((8, 64)),
            _full((1, HID)),
            _full((HID, C2)),
            _full((64, 16)),
            _full((64, 16)),
        ],
        out_specs=[
            pl.BlockSpec((_ROWBLK, ROWW), lambda i: (i, 0)),
            pl.BlockSpec((_ROWBLK, 16), lambda i: (i, 0)),
        ],
        out_shape=[
            jax.ShapeDtypeStruct((N, ROWW), f32),
            jax.ShapeDtypeStruct((N, 16), f32),
        ],
    )(acc1, erep, b1r, W2, a2sd, a2d)

    acc2 = _edge_kernel_l2(htab2, adtab2, src2d, dst2d)[:, :N]

    out = pl.pallas_call(
        _tc3_body,
        grid=(_GRID,),
        in_specs=[
            pl.BlockSpec((NC, _ROWBLK, ACCW), lambda i: (0, i, 0)),
            _full((8, 64)),
            _full((1, C2)),
        ],
        out_specs=pl.BlockSpec((_ROWBLK, C2), lambda i: (i, 0)),
        out_shape=jax.ShapeDtypeStruct((N, C2), f32),
    )(acc2, bmat, b2r)

    return out


# speedup vs baseline: 132.0127x; 2.4286x over previous
import functools

import numpy as np

import jax
import jax.numpy as jnp
from jax import lax
from jax.experimental import pallas as pl
from jax.experimental.pallas import tpu as pltpu
from jax.experimental.pallas import tpu_sc as plsc

N = 10000
E = 320000
D = 128
HID = 64
C2 = 64

NC = 2
NS = 16
NW = NC * NS
EPW = E // NW
SUB = 125
NSUB = 2
CHUNK = SUB * NSUB
NCHUNK = EPW // CHUNK
ROWW = 80
MC = 64
ACCW = 72
NPAD = 10112
NPT = NPAD // NS
NROWZ = 79

_ROWBLK = 1000
_GRID = N // _ROWBLK


def _make_edge_kernel(nheads):
    mesh = plsc.VectorSubcoreMesh(core_axis_name="c", subcore_axis_name="s")

    @functools.partial(
        pl.kernel,
        out_type=jax.ShapeDtypeStruct((NC, NPAD, ACCW), jnp.float32),
        mesh=mesh,
        scratch_types=(
            pltpu.VMEM((NSUB, SUB), jnp.int32),
            pltpu.VMEM((NSUB, SUB), jnp.int32),
            pltpu.VMEM((NSUB, SUB), jnp.int32),
            pltpu.VMEM((NSUB, SUB), jnp.int32),
            pltpu.VMEM((CHUNK, ROWW), jnp.float32),
            pltpu.VMEM((CHUNK, ROWW), jnp.float32),
            pltpu.VMEM((CHUNK, 16), jnp.float32),
            pltpu.VMEM((CHUNK, 16), jnp.float32),
            pltpu.VMEM((CHUNK, ACCW), jnp.float32),
            pltpu.VMEM_SHARED((NPAD, ACCW), jnp.float32),
            pltpu.SemaphoreType.DMA,
            pltpu.SemaphoreType.DMA,
            pltpu.SemaphoreType.DMA,
            pltpu.SemaphoreType.DMA,
        ),
        compiler_params=pltpu.CompilerParams(use_tc_tiling_on_sc=False,
                                             needs_layout_passes=False),
    )
    def edge_kernel(htab, adtab, src_i, dst_i, acc_o,
                    srcv0, srcv1, dstv0, dstv1, g1a, g1b, g2a, g2b,
                    msgex, acc_s, s1a, s1b, s2a, s2b):
        c = lax.axis_index("c")
        s = lax.axis_index("s")
        wid = s * NC + c
        z16 = jnp.zeros((16,), jnp.float32)
        bufs = ((srcv0, dstv0, g1a, g2a, s1a, s2a),
                (srcv1, dstv1, g1b, g2b, s1b, s2b))

        def fire(ch, b):
            si, di, g1, g2, sh, sa = bufs[b]
            rowbase = wid * (EPW // SUB) + ch * NSUB
            pltpu.sync_copy(src_i.at[pl.ds(rowbase, NSUB)], si)
            pltpu.sync_copy(dst_i.at[pl.ds(rowbase, NSUB)], di)
            for j in range(NSUB):
                pltpu.async_copy(htab.at[si.at[j]],
                                 g1.at[pl.ds(j * SUB, SUB)], sh)
                pltpu.async_copy(adtab.at[di.at[j]],
                                 g2.at[pl.ds(j * SUB, SUB)], sa)

        def drain(b):
            si, di, g1, g2, sh, sa = bufs[b]
            for j in range(NSUB):
                pltpu.make_async_copy(htab.at[si.at[j]],
                                      g1.at[pl.ds(j * SUB, SUB)], sh).wait()
                pltpu.make_async_copy(adtab.at[di.at[j]],
                                      g2.at[pl.ds(j * SUB, SUB)], sa).wait()

        def zrow(r, carry):
            for k in (0, 16, 32, 48, 56):
                msgex[r, pl.ds(k, 16)] = z16
            return carry

        lax.fori_loop(0, NROWZ, zrow, 0)
        for t in range(NPT // NROWZ):
            pltpu.sync_copy(msgex.at[pl.ds(0, NROWZ)],
                            acc_s.at[pl.ds(s * NPT + t * NROWZ, NROWZ)])
        plsc.subcore_barrier()

        gdn = lax.GatherDimensionNumbers(
            offset_dims=(), collapsed_slice_dims=(0,), start_index_map=(0,))

        def compute_scatter(b):
            si, di, g1, g2, sh, sa = bufs[b]

            @plsc.parallel_loop(0, CHUNK, unroll=4)
            def edge_body(e):
                av = g1[e, pl.ds(MC, 16)]
                bv = g2[e, pl.ds(0, 16)]
                es = av + bv
                es = jnp.maximum(es, es * jnp.float32(0.2))
                ex = jnp.exp(es)
                erow = jnp.full((16,), e, jnp.int32)
                iotl = lax.iota(jnp.int32, 16)
                plsc.store_scatter(msgex, [erow, MC + (iotl % 8)], ex,
                                   mask=iotl < 8)
                for v in range(MC // 16):
                    if nheads == 8:
                        pv = 2 * v + (iotl // 8)
                    else:
                        pv = 0 * (iotl // 8)
                    hx = g1[e, pl.ds(16 * v, 16)]
                    exv = lax.gather(
                        ex, pv[:, None], gdn, (1,),
                        mode=lax.GatherScatterMode.PROMISE_IN_BOUNDS)
                    msgex[e, pl.ds(16 * v, 16)] = hx * exv
            for j in range(NSUB):
                pltpu.sync_copy(msgex.at[pl.ds(j * SUB, SUB)],
                                acc_s.at[di.at[j]], add=True)

        fire(0, 0)

        def pair_body(i, carry):
            fire(2 * i + 1, 1)
            drain(0)
            compute_scatter(0)

            @pl.when(i < NCHUNK // 2 - 1)
            def _():
                fire(2 * i + 2, 0)

            drain(1)
            compute_scatter(1)
            return carry

        lax.fori_loop(0, NCHUNK // 2, pair_body, 0)
        plsc.subcore_barrier()

        pltpu.sync_copy(acc_s.at[pl.ds(s * NPT, NPT)],
                        acc_o.at[c, pl.ds(s * NPT, NPT)])

    return edge_kernel


_edge_kernel_l1 = _make_edge_kernel(8)
_edge_kernel_l2 = _make_edge_kernel(1)


def _tc1_body(x_ref, w1_ref, asd_ref, ad_ref, htab_ref, adtab_ref):
    h = jnp.dot(x_ref[...], w1_ref[...], preferred_element_type=jnp.float32)
    sa = jnp.dot(h, asd_ref[...], preferred_element_type=jnp.float32)
    htab_ref[...] = jnp.concatenate([h, sa], axis=1)
    adtab_ref[...] = jnp.dot(h, ad_ref[...], preferred_element_type=jnp.float32)


def _tc2_body(acc_ref, erep_ref, b1_ref, w2_ref, a2sd_ref, a2d_ref,
              htab_ref, adtab_ref):
    both = acc_ref[0] + acc_ref[1]
    acc = both[:, :MC]
    den = both[:, MC:ACCW]
    dex = jnp.dot(den, erep_ref[...], preferred_element_type=jnp.float32)
    h1 = acc / (dex + 1e-16) + b1_ref[...]
    h1 = jnp.where(h1 > 0, h1, jnp.exp(jnp.minimum(h1, 0.0)) - 1.0)
    h2 = jnp.dot(h1, w2_ref[...], preferred_element_type=jnp.float32)
    sa = jnp.dot(h2, a2sd_ref[...], preferred_element_type=jnp.float32)
    htab_ref[...] = jnp.concatenate([h2, sa], axis=1)
    adtab_ref[...] = jnp.dot(h2, a2d_ref[...], preferred_element_type=jnp.float32)


def _tc3_body(acc_ref, bmat_ref, b2_ref, out_ref):
    both = acc_ref[0] + acc_ref[1]
    acc = both[:, :MC]
    den = both[:, MC:ACCW]
    dex = jnp.dot(den, bmat_ref[...], preferred_element_type=jnp.float32)
    v = acc / (dex + 1e-16) + b2_ref[...]
    m = jnp.max(v, axis=1, keepdims=True)
    z = v - m
    out_ref[...] = z - jnp.log(jnp.sum(jnp.exp(z), axis=1, keepdims=True))


def _full(shape):
    return pl.BlockSpec(shape, lambda i: tuple(0 for _ in shape))


def kernel(x, edge_index, W1, a_src1, a_dst1, b1, W2, a_src2, a_dst2, b2):
    f32 = jnp.float32
    src2d = edge_index[0].reshape(E // SUB, SUB)
    dst2d = edge_index[1].reshape(E // SUB, SUB)

    eye8 = jnp.eye(8, dtype=f32)
    as64 = (a_src1[:, :, None] * eye8[:, None, :]).reshape(64, 8)
    ad64 = (a_dst1[:, :, None] * eye8[:, None, :]).reshape(64, 8)
    asd = jnp.concatenate([as64, ad64], axis=1)
    ad16 = jnp.concatenate([ad64, jnp.zeros((64, 8), f32)], axis=1)
    erep = jnp.repeat(jnp.eye(8, dtype=f32), 8, axis=1)
    a2sd = jnp.concatenate([a_src2.T, jnp.zeros((64, 15), f32)], axis=1)
    a2d = jnp.concatenate([a_dst2.T, jnp.zeros((64, 15), f32)], axis=1)
    bmat = jnp.concatenate([jnp.ones((1, 64), f32),
                            jnp.zeros((7, 64), f32)], axis=0)
    b1r = b1.reshape(1, HID)
    b2r = b2.reshape(1, C2)

    htab1, adtab1 = pl.pallas_call(
        _tc1_body,
        grid=(_GRID,),
        in_specs=[
            pl.BlockSpec((_ROWBLK, D), lambda i: (i, 0)),
            _full((D, HID)),
            _full((64, 16)),
            _full((64, 16)),
        ],
        out_specs=[
            pl.BlockSpec((_ROWBLK, ROWW), lambda i: (i, 0)),
            pl.BlockSpec((_ROWBLK, 16), lambda i: (i, 0)),
        ],
        out_shape=[
            jax.ShapeDtypeStruct((N, ROWW), f32),
            jax.ShapeDtypeStruct((N, 16), f32),
        ],
    )(x, W1, asd, ad16)

    acc1 = _edge_kernel_l1(htab1, adtab1, src2d, dst2d)[:, :N]

    htab2, adtab2 = pl.pallas_call(
        _tc2_body,
        grid=(_GRID,),
        in_specs=[
            pl.BlockSpec((NC, _ROWBLK, ACCW), lambda i: (0, i, 0)),
            _full((8, 64)),
            _full((1, HID)),
            _full((HID, C2)),
            _full((64, 16)),
            _full((64, 16)),
        ],
        out_specs=[
            pl.BlockSpec((_ROWBLK, ROWW), lambda i: (i, 0)),
            pl.BlockSpec((_ROWBLK, 16), lambda i: (i, 0)),
        ],
        out_shape=[
            jax.ShapeDtypeStruct((N, ROWW), f32),
            jax.ShapeDtypeStruct((N, 16), f32),
        ],
    )(acc1, erep, b1r, W2, a2sd, a2d)

    acc2 = _edge_kernel_l2(htab2, adtab2, src2d, dst2d)[:, :N]

    out = pl.pallas_call(
        _tc3_body,
        grid=(_GRID,),
        in_specs=[
            pl.BlockSpec((NC, _ROWBLK, ACCW), lambda i: (0, i, 0)),
            _full((8, 64)),
            _full((1, C2)),
        ],
        out_specs=pl.BlockSpec((_ROWBLK, C2), lambda i: (i, 0)),
        out_shape=jax.ShapeDtypeStruct((N, C2), f32),
    )(acc2, bmat, b2r)

    return out


# speedup vs baseline: 137.2667x; 1.0398x over previous
import functools

import numpy as np

import jax
import jax.numpy as jnp
from jax import lax
from jax.experimental import pallas as pl
from jax.experimental.pallas import tpu as pltpu
from jax.experimental.pallas import tpu_sc as plsc

N = 10000
E = 320000
D = 128
HID = 64
C2 = 64

NC = 2
NS = 16
NW = NC * NS
EPW = E // NW
SUB = 125
NSUB = 2
CHUNK = SUB * NSUB
NCHUNK = EPW // CHUNK
ROWW = 80
MC = 64
ACCW = 72
NPAD = 10112
NPT = NPAD // NS
NROWZ = 79

_ROWBLK = 1000
_GRID = N // _ROWBLK


def _make_edge_kernel(nheads):
    mesh = plsc.VectorSubcoreMesh(core_axis_name="c", subcore_axis_name="s")

    @functools.partial(
        pl.kernel,
        out_type=jax.ShapeDtypeStruct((NC, NPAD, ACCW), jnp.float32),
        mesh=mesh,
        scratch_types=(
            pltpu.VMEM((NSUB, SUB), jnp.int32),
            pltpu.VMEM((NSUB, SUB), jnp.int32),
            pltpu.VMEM((NSUB, SUB), jnp.int32),
            pltpu.VMEM((NSUB, SUB), jnp.int32),
            pltpu.VMEM((CHUNK, ROWW), jnp.float32),
            pltpu.VMEM((CHUNK, ROWW), jnp.float32),
            pltpu.VMEM((CHUNK, 16), jnp.float32),
            pltpu.VMEM((CHUNK, 16), jnp.float32),
            pltpu.VMEM((CHUNK, ACCW), jnp.float32),
            pltpu.VMEM_SHARED((NPAD, ACCW), jnp.float32),
            pltpu.SemaphoreType.DMA,
            pltpu.SemaphoreType.DMA,
            pltpu.SemaphoreType.DMA,
            pltpu.SemaphoreType.DMA,
        ),
        compiler_params=pltpu.CompilerParams(use_tc_tiling_on_sc=False,
                                             needs_layout_passes=False),
    )
    def edge_kernel(htab, adtab, src_i, dst_i, acc_o,
                    srcv0, srcv1, dstv0, dstv1, g1a, g1b, g2a, g2b,
                    msgex, acc_s, s1a, s1b, s2a, s2b):
        c = lax.axis_index("c")
        s = lax.axis_index("s")
        wid = s * NC + c
        z16 = jnp.zeros((16,), jnp.float32)
        bufs = ((srcv0, dstv0, g1a, g2a, s1a, s2a),
                (srcv1, dstv1, g1b, g2b, s1b, s2b))

        def fire(ch, b):
            si, di, g1, g2, sh, sa = bufs[b]
            rowbase = wid * (EPW // SUB) + ch * NSUB
            pltpu.sync_copy(src_i.at[pl.ds(rowbase, NSUB)], si)
            pltpu.sync_copy(dst_i.at[pl.ds(rowbase, NSUB)], di)
            for j in range(NSUB):
                pltpu.async_copy(htab.at[si.at[j]],
                                 g1.at[pl.ds(j * SUB, SUB)], sh)
                pltpu.async_copy(adtab.at[di.at[j]],
                                 g2.at[pl.ds(j * SUB, SUB)], sa)

        def drain(b):
            si, di, g1, g2, sh, sa = bufs[b]
            for j in range(NSUB):
                pltpu.make_async_copy(htab.at[si.at[j]],
                                      g1.at[pl.ds(j * SUB, SUB)], sh).wait()
                pltpu.make_async_copy(adtab.at[di.at[j]],
                                      g2.at[pl.ds(j * SUB, SUB)], sa).wait()

        def zrow(r, carry):
            for k in (0, 16, 32, 48, 56):
                msgex[r, pl.ds(k, 16)] = z16
            return carry

        lax.fori_loop(0, NROWZ, zrow, 0)
        for t in range(NPT // NROWZ):
            pltpu.sync_copy(msgex.at[pl.ds(0, NROWZ)],
                            acc_s.at[pl.ds(s * NPT + t * NROWZ, NROWZ)])
        plsc.subcore_barrier()

        gdn = lax.GatherDimensionNumbers(
            offset_dims=(), collapsed_slice_dims=(0,), start_index_map=(0,))

        def compute_scatter(b):
            si, di, g1, g2, sh, sa = bufs[b]

            @plsc.parallel_loop(0, CHUNK, unroll=4)
            def edge_body(e):
                av = g1[e, pl.ds(MC, 16)]
                bv = g2[e, pl.ds(0, 16)]
                es = av + bv
                es = jnp.maximum(es, es * jnp.float32(0.2))
                ex = jnp.exp(es)
                erow = jnp.full((16,), e, jnp.int32)
                iotl = lax.iota(jnp.int32, 16)
                plsc.store_scatter(msgex, [erow, MC + (iotl % 8)], ex,
                                   mask=iotl < 8)
                for v in range(MC // 16):
                    if nheads == 8:
                        pv = 2 * v + (iotl // 8)
                    else:
                        pv = 0 * (iotl // 8)
                    hx = g1[e, pl.ds(16 * v, 16)]
                    exv = lax.gather(
                        ex, pv[:, None], gdn, (1,),
                        mode=lax.GatherScatterMode.PROMISE_IN_BOUNDS)
                    msgex[e, pl.ds(16 * v, 16)] = hx * exv
            for j in range(NSUB):
                pltpu.sync_copy(msgex.at[pl.ds(j * SUB, SUB)],
                                acc_s.at[di.at[j]], add=True)

        fire(0, 0)

        def pair_body(i, carry):
            fire(2 * i + 1, 1)
            drain(0)
            compute_scatter(0)

            @pl.when(i < NCHUNK // 2 - 1)
            def _():
                fire(2 * i + 2, 0)

            drain(1)
            compute_scatter(1)
            return carry

        lax.fori_loop(0, NCHUNK // 2, pair_body, 0)
        plsc.subcore_barrier()

        pltpu.sync_copy(acc_s.at[pl.ds(s * NPT, NPT)],
                        acc_o.at[c, pl.ds(s * NPT, NPT)])

    return edge_kernel


_edge_kernel_l1 = _make_edge_kernel(8)
_edge_kernel_l2 = _make_edge_kernel(1)


def _tc1_body(x_ref, w1_ref, asd_ref, ad_ref, htab_ref, adtab_ref):
    h = jnp.dot(x_ref[...], w1_ref[...], preferred_element_type=jnp.float32)
    sa = jnp.dot(h, asd_ref[...], preferred_element_type=jnp.float32)
    htab_ref[...] = jnp.concatenate([h, sa], axis=1)
    adtab_ref[...] = jnp.dot(h, ad_ref[...], preferred_element_type=jnp.float32)


def _tc2_body(acc_ref, erep_ref, b1_ref, w2_ref, a2sd_ref, a2d_ref,
              htab_ref, adtab_ref):
    both = acc_ref[0] + acc_ref[1]
    acc = both[:, :MC]
    den = both[:, MC:ACCW]
    dex = jnp.dot(den, erep_ref[...], preferred_element_type=jnp.float32)
    h1 = acc / (dex + 1e-16) + b1_ref[...]
    h1 = jnp.where(h1 > 0, h1, jnp.exp(jnp.minimum(h1, 0.0)) - 1.0)
    h2 = jnp.dot(h1, w2_ref[...], preferred_element_type=jnp.float32)
    sa = jnp.dot(h2, a2sd_ref[...], preferred_element_type=jnp.float32)
    htab_ref[...] = jnp.concatenate([h2, sa], axis=1)
    adtab_ref[...] = jnp.dot(h2, a2d_ref[...], preferred_element_type=jnp.float32)


def _tc3_body(acc_ref, bmat_ref, b2_ref, out_ref):
    both = acc_ref[0] + acc_ref[1]
    acc = both[:, :MC]
    den = both[:, MC:ACCW]
    dex = jnp.dot(den, bmat_ref[...], preferred_element_type=jnp.float32)
    v = acc / (dex + 1e-16) + b2_ref[...]
    m = jnp.max(v, axis=1, keepdims=True)
    z = v - m
    out_ref[...] = z - jnp.log(jnp.sum(jnp.exp(z), axis=1, keepdims=True))


def _full(shape):
    return pl.BlockSpec(shape, lambda i: tuple(0 for _ in shape))


def kernel(x, edge_index, W1, a_src1, a_dst1, b1, W2, a_src2, a_dst2, b2):
    f32 = jnp.float32
    src2d = edge_index[0].reshape(E // SUB, SUB)
    dst2d = edge_index[1].reshape(E // SUB, SUB)

    eye8 = jnp.eye(8, dtype=f32)
    as64 = (a_src1[:, :, None] * eye8[:, None, :]).reshape(64, 8)
    ad64 = (a_dst1[:, :, None] * eye8[:, None, :]).reshape(64, 8)
    asd = jnp.concatenate([as64, ad64], axis=1)
    ad16 = jnp.concatenate([ad64, jnp.zeros((64, 8), f32)], axis=1)
    erep = jnp.repeat(jnp.eye(8, dtype=f32), 8, axis=1)
    a2sd = jnp.concatenate([a_src2.T, jnp.zeros((64, 15), f32)], axis=1)
    a2d = jnp.concatenate([a_dst2.T, jnp.zeros((64, 15), f32)], axis=1)
    bmat = jnp.concatenate([jnp.ones((1, 64), f32),
                            jnp.zeros((7, 64), f32)], axis=0)
    b1r = b1.reshape(1, HID)
    b2r = b2.reshape(1, C2)

    htab1, adtab1 = pl.pallas_call(
        _tc1_body,
        grid=(_GRID,),
        in_specs=[
            pl.BlockSpec((_ROWBLK, D), lambda i: (i, 0)),
            _full((D, HID)),
            _full((64, 16)),
            _full((64, 16)),
        ],
        out_specs=[
            pl.BlockSpec((_ROWBLK, ROWW), lambda i: (i, 0)),
            pl.BlockSpec((_ROWBLK, 16), lambda i: (i, 0)),
        ],
        out_shape=[
            jax.ShapeDtypeStruct((N, ROWW), f32),
            jax.ShapeDtypeStruct((N, 16), f32),
        ],
    )(x, W1, asd, ad16)

    acc1 = _edge_kernel_l1(htab1, adtab1, src2d, dst2d)

    htab2, adtab2 = pl.pallas_call(
        _tc2_body,
        grid=(_GRID,),
        in_specs=[
            pl.BlockSpec((NC, _ROWBLK, ACCW), lambda i: (0, i, 0)),
            _full((8, 64)),
            _full((1, HID)),
            _full((HID, C2)),
            _full((64, 16)),
            _full((64, 16)),
        ],
        out_specs=[
            pl.BlockSpec((_ROWBLK, ROWW), lambda i: (i, 0)),
            pl.BlockSpec((_ROWBLK, 16), lambda i: (i, 0)),
        ],
        out_shape=[
            jax.ShapeDtypeStruct((N, ROWW), f32),
            jax.ShapeDtypeStruct((N, 16), f32),
        ],
    )(acc1, erep, b1r, W2, a2sd, a2d)

    acc2 = _edge_kernel_l2(htab2, adtab2, src2d, dst2d)

    out = pl.pallas_call(
        _tc3_body,
        grid=(_GRID,),
        in_specs=[
            pl.BlockSpec((NC, _ROWBLK, ACCW), lambda i: (0, i, 0)),
            _full((8, 64)),
            _full((1, C2)),
        ],
        out_specs=pl.BlockSpec((_ROWBLK, C2), lambda i: (i, 0)),
        out_shape=jax.ShapeDtypeStruct((N, C2), f32),
    )(acc2, bmat, b2r)

    return out
